# Initial kernel scaffold; baseline (speedup 1.0000x reference)
import jax
import jax.numpy as jnp
from jax.experimental import pallas as pl


def kernel(pos, edge_index, batch, z, params):
    raise NotImplementedError("write your pallas kernel here")


# speedup vs baseline: 3.9073x; 3.9073x over previous
import functools

import jax
import jax.numpy as jnp
from jax import lax
from jax.experimental import pallas as pl
from jax.experimental.pallas import tpu as pltpu
from jax.experimental.pallas import tpu_sc as plsc

N = 10000
E = 320000
CHN = 128
EPS = 1e-5
BLK = 2000
GRID = N // BLK

_NC, _NS = 2, 16
_NW = _NC * _NS
_EPT = E // _NW
_K = 80
_NCH = _EPT // _K
_NPAD = 10240
_RPT = _NPAD // _NS
_ZR = 128


def _leaky(x, s):
    return jnp.where(x > 0, x, s * x)


def _edge_body(a_hbm, c_hbm, src_hbm, dst_hbm, out_hbm,
               srcv, dstv, av, cv, zv, acc, sem_a, sem_c):
    cid = lax.axis_index("c")
    sid = lax.axis_index("s")
    wid = cid * _NS + sid

    def zrow(j, _):
        r = j // 8
        d = (j % 8) * 16
        zv[r, pl.ds(d, 16)] = jnp.zeros((16,), jnp.float32)
        return 0
    lax.fori_loop(0, _ZR * 8, zrow, 0)

    def zcp(j, _):
        pltpu.sync_copy(zv, acc.at[pl.ds(sid * _RPT + j * _ZR, _ZR)])
        return 0
    lax.fori_loop(0, _RPT // _ZR, zcp, 0)
    plsc.subcore_barrier()

    def chunk(i, _):
        base = wid * _EPT + i * _K
        pltpu.sync_copy(src_hbm.at[pl.ds(base, _K)], srcv)
        pltpu.sync_copy(dst_hbm.at[pl.ds(base, _K)], dstv)
        cp_a = pltpu.async_copy(a_hbm.at[srcv], av, sem_a)
        cp_c = pltpu.async_copy(c_hbm.at[dstv], cv, sem_c)
        cp_a.wait()
        cp_c.wait()

        def ebody(j, _):
            e = j // 8
            d = (j % 8) * 16
            a = av[e, pl.ds(d, 16)]
            c = cv[e, pl.ds(d, 16)]
            cv[e, pl.ds(d, 16)] = jnp.maximum(a + c, 0.0)
            return 0
        lax.fori_loop(0, _K * 8, ebody, 0)

        pltpu.sync_copy(cv, acc.at[dstv], add=True)
        return 0
    lax.fori_loop(0, _NCH, chunk, 0)
    plsc.subcore_barrier()

    def wcp(j, _):
        r0 = sid * _RPT + j * _ZR
        pltpu.sync_copy(acc.at[pl.ds(r0, _ZR)],
                        out_hbm.at[cid, pl.ds(r0, _ZR)])
        return 0
    lax.fori_loop(0, _RPT // _ZR, wcp, 0)


@functools.cache
def _edge_agg_fn():
    return pl.kernel(
        _edge_body,
        out_type=jax.ShapeDtypeStruct((2, _NPAD, CHN), jnp.float32),
        mesh=plsc.VectorSubcoreMesh(core_axis_name="c", subcore_axis_name="s",
                                    num_cores=_NC, num_subcores=_NS),
        scratch_types=[
            pltpu.VMEM((_K,), jnp.int32),
            pltpu.VMEM((_K,), jnp.int32),
            pltpu.VMEM((_K, CHN), jnp.float32),
            pltpu.VMEM((_K, CHN), jnp.float32),
            pltpu.VMEM((_ZR, CHN), jnp.float32),
            pltpu.VMEM_SHARED((_NPAD, CHN), jnp.float32),
            pltpu.SemaphoreType.DMA,
            pltpu.SemaphoreType.DMA,
        ],
    )


def _edge_agg(a, c, src, dst):
    parts = _edge_agg_fn()(a, c, src, dst)
    return parts[0, :N], parts[1, :N]


def _row_spec(cols):
    return pl.BlockSpec((BLK, cols), lambda i: (i, 0))


def _full_spec(shape):
    return pl.BlockSpec(shape, lambda i: (0,) * len(shape))


def _ada_update(x, p0, p1, style, wg0, bg0, wg1, bg1, wag, bag, wab, bab):
    agg = p0 + p1
    t = jnp.maximum(agg @ wg0 + bg0, 0.0)
    o = x + t @ wg1 + bg1
    o = _leaky(o, 0.2)
    gamma = style @ wag + bag
    beta = style @ wab + bab
    mu = jnp.mean(o, axis=1, keepdims=True)
    var = jnp.mean((o - mu) ** 2, axis=1, keepdims=True)
    xn = (o - mu) * lax.rsqrt(var + EPS)
    return gamma * xn + beta


def _hdelta_ac(x, pos, wh0, bh0, wh1, bh1, wfp, wfx, bf):
    h = jnp.maximum(x @ wh0 + bh0, 0.0)
    delta = jnp.tanh(h @ wh1 + bh1)
    a = pos @ wfp + x @ wfx
    c = (delta - pos) @ wfp + bf
    return a, c


def _tc1_body(pos_r, z_r, w0z, w0p, b0, w1, b1, brcat, phase,
              wh0, bh0, wh1, bh1, wfp, wfx, bf,
              style_o, x0_o, a_o, c_o):
    pos = pos_r[...]
    z = z_r[...]
    s = _leaky(z @ w0z[...] + pos @ w0p[...] + b0[...], 0.01)
    style = _leaky(s @ w1[...] + b1[...], 0.01)
    style_o[...] = style
    v = ((2.0 * jnp.pi) * pos) @ brcat[...] - phase[...]
    c1 = jnp.float32(6.2831855)
    c2 = jnp.float32(-1.7484555e-07)
    k = jnp.round(v * jnp.float32(1.0 / (2.0 * jnp.pi)))
    r = (v - k * c1) - k * c2
    x0 = jnp.cos(r)
    x0_o[...] = x0
    a, c = _hdelta_ac(x0, pos, wh0[...], bh0[...], wh1[...], bh1[...],
                      wfp[...], wfx[...], bf[...])
    a_o[...] = a
    c_o[...] = c


def _tc2_body(x_r, p0_r, p1_r, style_r, pos_r,
              wg0, bg0, wg1, bg1, wag, bag, wab, bab,
              wh0, bh0, wh1, bh1, wfp, wfx, bf,
              x1_o, a_o, c_o):
    x1 = _ada_update(x_r[...], p0_r[...], p1_r[...], style_r[...],
                     wg0[...], bg0[...], wg1[...], bg1[...],
                     wag[...], bag[...], wab[...], bab[...])
    x1_o[...] = x1
    a, c = _hdelta_ac(x1, pos_r[...], wh0[...], bh0[...], wh1[...], bh1[...],
                      wfp[...], wfx[...], bf[...])
    a_o[...] = a
    c_o[...] = c


def _tc3a_body(x_r, p0_r, p1_r, style_r, batch_r,
               wg0, bg0, wg1, bg1, wag, bag, wab, bab,
               x2_o, gmax_o):
    x2 = _ada_update(x_r[...], p0_r[...], p1_r[...], style_r[...],
                     wg0[...], bg0[...], wg1[...], bg1[...],
                     wag[...], bag[...], wab[...], bab[...])
    x2_o[...] = x2
    b = batch_r[...]
    neg = jnp.full_like(x2, -jnp.inf)
    g0 = jnp.max(jnp.where(b == 0, x2, neg), axis=0, keepdims=True)
    g1 = jnp.max(jnp.where(b == 1, x2, neg), axis=0, keepdims=True)
    cur = jnp.concatenate([g0, g1], axis=0)
    i = pl.program_id(0)

    @pl.when(i == 0)
    def _():
        gmax_o[...] = cur

    @pl.when(i != 0)
    def _():
        gmax_o[...] = jnp.maximum(gmax_o[...], cur)


def _tc3b_body(x2_r, z_r, batch_r, gmax_r,
               wgc0, bgc0, wgc1, bgc1,
               wt0x, wt0g, bt0, wt1, bt1, wt2, bt2,
               w0zg, w0pg, b0g, w1g, b1g,
               wfex, wfeg, bfe,
               wh0, bh0, wh1, bh1, wfp, wfx, bf,
               pc_o, h0_o, styleg_o, a_o, c_o):
    x2 = x2_r[...]
    z = z_r[...]
    b = batch_r[...]
    gg = _leaky(gmax_r[...] @ wgc0[...] + bgc0[...], 0.01)
    gg = _leaky(gg @ wgc1[...] + bgc1[...], 0.01)
    gsel = jnp.where(b == 0, gg[0:1, :], gg[1:2, :])
    t = _leaky(x2 @ wt0x[...] + gsel @ wt0g[...] + bt0[...], 0.01)
    t = _leaky(t @ wt1[...] + bt1[...], 0.01)
    pc = jnp.tanh(t @ wt2[...] + bt2[...]) * 0.75
    pc_o[...] = pc
    sg = _leaky(z @ w0zg[...] + pc @ w0pg[...] + b0g[...], 0.01)
    styleg = _leaky(sg @ w1g[...] + b1g[...], 0.01)
    styleg_o[...] = styleg
    h0 = _leaky(x2 @ wfex[...] + gsel @ wfeg[...] + bfe[...], 0.01)
    h0_o[...] = h0
    a, c = _hdelta_ac(h0, pc, wh0[...], bh0[...], wh1[...], bh1[...],
                      wfp[...], wfx[...], bf[...])
    a_o[...] = a
    c_o[...] = c


def _tc4_body(h0_r, p0_r, p1_r, styleg_r,
              wg0, bg0, wg1, bg1, wag, bag, wab, bab,
              hout_o):
    hout_o[...] = _ada_update(h0_r[...], p0_r[...], p1_r[...], styleg_r[...],
                              wg0[...], bg0[...], wg1[...], bg1[...],
                              wag[...], bag[...], wab[...], bab[...])


def _wspecs(ws):
    return [_full_spec(w.shape) for w in ws]


def _blk_weights(bp):
    wh0 = bp['h0'][0].T
    bh0 = bp['h0'][1][None, :]
    wh1 = bp['h1'][0].T
    bh1 = bp['h1'][1][None, :]
    wf = bp['f0'][0]
    wfp = wf[:, :3].T
    wfx = wf[:, 3:].T
    bf = bp['f0'][1][None, :]
    wg0 = bp['g0'][0].T
    bg0 = bp['g0'][1][None, :]
    wg1 = bp['g1'][0].T
    bg1 = bp['g1'][1][None, :]
    wa = bp['ada'][0]
    ba = bp['ada'][1]
    wag = wa[:CHN].T
    bag = ba[None, :CHN]
    wab = wa[CHN:].T
    bab = ba[None, CHN:]
    return dict(wh0=wh0, bh0=bh0, wh1=wh1, bh1=bh1, wfp=wfp, wfx=wfx, bf=bf,
                wg0=wg0, bg0=bg0, wg1=wg1, bg1=bg1,
                wag=wag, bag=bag, wab=wab, bab=bab)


def kernel(pos, edge_index, batch, z, params):
    cp = params['cloud']
    gp = params['gauss']
    src = edge_index[0]
    dst = edge_index[1]
    batch2 = batch[:, None]

    w0 = cp['style0'][0]
    w0z, w0p = w0[:, :128].T, w0[:, 128:].T
    b0 = cp['style0'][1][None, :]
    w1 = cp['style1'][0].T
    b1 = cp['style1'][1][None, :]
    br = cp['Brff']
    brcat = jnp.concatenate([br.T, br.T], axis=1)
    phase = jnp.concatenate([jnp.zeros((1, 64), jnp.float32),
                             jnp.full((1, 64), 0.5 * jnp.pi, jnp.float32)],
                            axis=1)
    cb0 = _blk_weights(cp['blocks'][0])
    cb1 = _blk_weights(cp['blocks'][1])
    gb0 = _blk_weights(gp['blocks'][0])
    wgc0 = cp['gc0'][0].T
    bgc0 = cp['gc0'][1][None, :]
    wgc1 = cp['gc1'][0].T
    bgc1 = cp['gc1'][1][None, :]
    wt0 = cp['tail0'][0]
    wt0x, wt0g = wt0[:, :CHN].T, wt0[:, CHN:].T
    bt0 = cp['tail0'][1][None, :]
    wt1 = cp['tail1'][0].T
    bt1 = cp['tail1'][1][None, :]
    wt2 = cp['tail2'][0].T
    bt2 = cp['tail2'][1][None, :]
    wg0s = gp['style0'][0]
    w0zg, w0pg = wg0s[:, :128].T, wg0s[:, 128:].T
    b0g = gp['style0'][1][None, :]
    w1g = gp['style1'][0].T
    b1g = gp['style1'][1][None, :]
    wfe = gp['fe'][0]
    wfex, wfeg = wfe[:, :CHN].T, wfe[:, CHN:].T
    bfe = gp['fe'][1][None, :]

    f32 = jnp.float32
    rows = lambda c: jax.ShapeDtypeStruct((N, c), f32)

    ws1 = [w0z, w0p, b0, w1, b1, brcat, phase,
           cb0['wh0'], cb0['bh0'], cb0['wh1'], cb0['bh1'],
           cb0['wfp'], cb0['wfx'], cb0['bf']]
    style, x0, a1, c1 = pl.pallas_call(
        _tc1_body,
        grid=(GRID,),
        in_specs=[_row_spec(3), _row_spec(CHN)] + _wspecs(ws1),
        out_specs=[_row_spec(CHN)] * 4,
        out_shape=[rows(CHN)] * 4,
    )(pos, z, *ws1)

    pt1a, pt1b = _edge_agg(a1, c1, src, dst)

    ws2 = [cb0['wg0'], cb0['bg0'], cb0['wg1'], cb0['bg1'],
           cb0['wag'], cb0['bag'], cb0['wab'], cb0['bab'],
           cb1['wh0'], cb1['bh0'], cb1['wh1'], cb1['bh1'],
           cb1['wfp'], cb1['wfx'], cb1['bf']]
    x1, a2, c2 = pl.pallas_call(
        _tc2_body,
        grid=(GRID,),
        in_specs=[_row_spec(CHN)] * 4 + [_row_spec(3)] + _wspecs(ws2),
        out_specs=[_row_spec(CHN)] * 3,
        out_shape=[rows(CHN)] * 3,
    )(x0, pt1a, pt1b, style, pos, *ws2)

    pt2a, pt2b = _edge_agg(a2, c2, src, dst)

    ws3a = [cb1['wg0'], cb1['bg0'], cb1['wg1'], cb1['bg1'],
            cb1['wag'], cb1['bag'], cb1['wab'], cb1['bab']]
    x2, gmax = pl.pallas_call(
        _tc3a_body,
        grid=(GRID,),
        in_specs=[_row_spec(CHN)] * 4 + [_row_spec(1)] + _wspecs(ws3a),
        out_specs=[_row_spec(CHN), _full_spec((2, CHN))],
        out_shape=[rows(CHN), jax.ShapeDtypeStruct((2, CHN), f32)],
    )(x1, pt2a, pt2b, style, batch2, *ws3a)

    ws3b = [wgc0, bgc0, wgc1, bgc1,
            wt0x, wt0g, bt0, wt1, bt1, wt2, bt2,
            w0zg, w0pg, b0g, w1g, b1g,
            wfex, wfeg, bfe,
            gb0['wh0'], gb0['bh0'], gb0['wh1'], gb0['bh1'],
            gb0['wfp'], gb0['wfx'], gb0['bf']]
    pc, h0, styleg, a3, c3 = pl.pallas_call(
        _tc3b_body,
        grid=(GRID,),
        in_specs=[_row_spec(CHN), _row_spec(CHN), _row_spec(1),
                  _full_spec((2, CHN))] + _wspecs(ws3b),
        out_specs=[_row_spec(3)] + [_row_spec(CHN)] * 4,
        out_shape=[rows(3)] + [rows(CHN)] * 4,
    )(x2, z, batch2, gmax, *ws3b)

    pt3a, pt3b = _edge_agg(a3, c3, src, dst)

    ws4 = [gb0['wg0'], gb0['bg0'], gb0['wg1'], gb0['bg1'],
           gb0['wag'], gb0['bag'], gb0['wab'], gb0['bab']]
    hout = pl.pallas_call(
        _tc4_body,
        grid=(GRID,),
        in_specs=[_row_spec(CHN)] * 4 + _wspecs(ws4),
        out_specs=_row_spec(CHN),
        out_shape=rows(CHN),
    )(h0, pt3a, pt3b, styleg, *ws4)

    return jnp.concatenate([pc, hout], axis=-1)


# speedup vs baseline: 4.8636x; 1.2448x over previous
import functools

import jax
import jax.numpy as jnp
from jax import lax
from jax.experimental import pallas as pl
from jax.experimental.pallas import tpu as pltpu
from jax.experimental.pallas import tpu_sc as plsc

N = 10000
E = 320000
CHN = 128
EPS = 1e-5
BLK = 2000
GRID = N // BLK

_NC, _NS = 2, 16
_NW = _NC * _NS
_EPT = E // _NW
_K = 80
_NCH = _EPT // _K
_NPAD = 10240
_RPT = _NPAD // _NS


def _leaky(x, s):
    return jnp.where(x > 0, x, s * x)


_NBUF = 2
_NIDX = 4


def _edge_body(a_hbm, c_hbm, src_hbm, dst_hbm, out_hbm,
               si0, si1, si2, si3, di0, di1, di2, di3,
               av0, av1, cv0, cv1, acc,
               is0, is1, is2, is3, sa0, sa1, sc0, sc1):
    sis = [si0, si1, si2, si3]
    dis = [di0, di1, di2, di3]
    iss = [is0, is1, is2, is3]
    avs = [av0, av1]
    cvs = [cv0, cv1]
    sas = [sa0, sa1]
    scs = [sc0, sc1]
    cid = lax.axis_index("c")
    sid = lax.axis_index("s")
    wid = cid * _NS + sid

    def idx_load(chunk, q):
        base = wid * _EPT + chunk * _K
        pltpu.sync_copy(src_hbm.at[pl.ds(base, _K)], sis[q])
        pltpu.sync_copy(dst_hbm.at[pl.ds(base, _K)], dis[q])

    def zrow(j, _):
        e = j // 8
        d = (j % 8) * 16
        av0[e, pl.ds(d, 16)] = jnp.zeros((16,), jnp.float32)
        return 0
    lax.fori_loop(0, _K * 8, zrow, 0)

    def zcp(j, _):
        pltpu.sync_copy(av0, acc.at[pl.ds(sid * _RPT + j * _K, _K)])
        return 0
    lax.fori_loop(0, _RPT // _K, zcp, 0)

    for b in range(_NBUF):
        idx_load(b, b)
        pltpu.async_copy(a_hbm.at[sis[b]], avs[b], sas[b])
        pltpu.async_copy(c_hbm.at[dis[b]], cvs[b], scs[b])
    plsc.subcore_barrier()

    def slot(i, k):
        b = k % _NBUF
        q = k
        q2 = (k + _NBUF) % _NIDX
        pltpu.make_async_copy(a_hbm.at[sis[q]], avs[b], sas[b]).wait()
        pltpu.make_async_copy(c_hbm.at[dis[q]], cvs[b], scs[b]).wait()

        def ebody(j, _):
            e = j // 8
            d = (j % 8) * 16
            a = avs[b][e, pl.ds(d, 16)]
            c = cvs[b][e, pl.ds(d, 16)]
            cvs[b][e, pl.ds(d, 16)] = jnp.maximum(a + c, 0.0)
            return 0
        lax.fori_loop(0, _K * 8, ebody, 0)

        pltpu.sync_copy(cvs[b], acc.at[dis[q]], add=True)

        @pl.when(i + _NBUF < _NCH)
        def _():
            idx_load(i + _NBUF, q2)
            pltpu.async_copy(a_hbm.at[sis[q2]], avs[b], sas[b])
            pltpu.async_copy(c_hbm.at[dis[q2]], cvs[b], scs[b])


    def outer(g, _):
        for k in range(_NIDX):
            i = g * _NIDX + k

            @pl.when(i < _NCH)
            def _():
                slot(i, k)
        return 0
    lax.fori_loop(0, (_NCH + _NIDX - 1) // _NIDX, outer, 0)
    plsc.subcore_barrier()
    pltpu.sync_copy(acc.at[pl.ds(sid * _RPT, _RPT)],
                    out_hbm.at[cid, pl.ds(sid * _RPT, _RPT)])


@functools.cache
def _edge_agg_fn():
    return pl.kernel(
        _edge_body,
        out_type=jax.ShapeDtypeStruct((2, _NPAD, CHN), jnp.float32),
        mesh=plsc.VectorSubcoreMesh(core_axis_name="c", subcore_axis_name="s",
                                    num_cores=_NC, num_subcores=_NS),
        scratch_types=(
            [pltpu.VMEM((_K,), jnp.int32)] * (2 * _NIDX)
            + [pltpu.VMEM((_K, CHN), jnp.float32)] * (2 * _NBUF)
            + [pltpu.VMEM_SHARED((_NPAD, CHN), jnp.float32)]
            + [pltpu.SemaphoreType.DMA] * (_NIDX + 2 * _NBUF)
        ),
    )


def _edge_agg(a, c, src, dst):
    parts = _edge_agg_fn()(a, c, src, dst)
    return parts[0, :N], parts[1, :N]


def _row_spec(cols):
    return pl.BlockSpec((BLK, cols), lambda i: (i, 0))


def _full_spec(shape):
    return pl.BlockSpec(shape, lambda i: (0,) * len(shape))


def _ada_update(x, p0, p1, style, wg0, bg0, wg1, bg1, wag, bag, wab, bab):
    agg = p0 + p1
    t = jnp.maximum(agg @ wg0 + bg0, 0.0)
    o = x + t @ wg1 + bg1
    o = _leaky(o, 0.2)
    gamma = style @ wag + bag
    beta = style @ wab + bab
    mu = jnp.mean(o, axis=1, keepdims=True)
    var = jnp.mean((o - mu) ** 2, axis=1, keepdims=True)
    xn = (o - mu) * lax.rsqrt(var + EPS)
    return gamma * xn + beta


def _hdelta_ac(x, pos, wh0, bh0, wh1, bh1, wfp, wfx, bf):
    h = jnp.maximum(x @ wh0 + bh0, 0.0)
    delta = jnp.tanh(h @ wh1 + bh1)
    a = pos @ wfp + x @ wfx
    c = (delta - pos) @ wfp + bf
    return a, c


def _tc1_body(pos_r, z_r, w0z, w0p, b0, w1, b1, brcat, phase,
              wh0, bh0, wh1, bh1, wfp, wfx, bf,
              style_o, x0_o, a_o, c_o):
    pos = pos_r[...]
    z = z_r[...]
    s = _leaky(z @ w0z[...] + pos @ w0p[...] + b0[...], 0.01)
    style = _leaky(s @ w1[...] + b1[...], 0.01)
    style_o[...] = style
    v = ((2.0 * jnp.pi) * pos) @ brcat[...] - phase[...]
    c1 = jnp.float32(6.2831855)
    c2 = jnp.float32(-1.7484555e-07)
    k = jnp.round(v * jnp.float32(1.0 / (2.0 * jnp.pi)))
    r = (v - k * c1) - k * c2
    x0 = jnp.cos(r)
    x0_o[...] = x0
    a, c = _hdelta_ac(x0, pos, wh0[...], bh0[...], wh1[...], bh1[...],
                      wfp[...], wfx[...], bf[...])
    a_o[...] = a
    c_o[...] = c


def _tc2_body(x_r, p0_r, p1_r, style_r, pos_r,
              wg0, bg0, wg1, bg1, wag, bag, wab, bab,
              wh0, bh0, wh1, bh1, wfp, wfx, bf,
              x1_o, a_o, c_o):
    x1 = _ada_update(x_r[...], p0_r[...], p1_r[...], style_r[...],
                     wg0[...], bg0[...], wg1[...], bg1[...],
                     wag[...], bag[...], wab[...], bab[...])
    x1_o[...] = x1
    a, c = _hdelta_ac(x1, pos_r[...], wh0[...], bh0[...], wh1[...], bh1[...],
                      wfp[...], wfx[...], bf[...])
    a_o[...] = a
    c_o[...] = c


def _tc3a_body(x_r, p0_r, p1_r, style_r, batch_r,
               wg0, bg0, wg1, bg1, wag, bag, wab, bab,
               x2_o, gmax_o):
    x2 = _ada_update(x_r[...], p0_r[...], p1_r[...], style_r[...],
                     wg0[...], bg0[...], wg1[...], bg1[...],
                     wag[...], bag[...], wab[...], bab[...])
    x2_o[...] = x2
    b = batch_r[...]
    neg = jnp.full_like(x2, -jnp.inf)
    g0 = jnp.max(jnp.where(b == 0, x2, neg), axis=0, keepdims=True)
    g1 = jnp.max(jnp.where(b == 1, x2, neg), axis=0, keepdims=True)
    cur = jnp.concatenate([g0, g1], axis=0)
    i = pl.program_id(0)

    @pl.when(i == 0)
    def _():
        gmax_o[...] = cur

    @pl.when(i != 0)
    def _():
        gmax_o[...] = jnp.maximum(gmax_o[...], cur)


def _tc3b_body(x2_r, z_r, batch_r, gmax_r,
               wgc0, bgc0, wgc1, bgc1,
               wt0x, wt0g, bt0, wt1, bt1, wt2, bt2,
               w0zg, w0pg, b0g, w1g, b1g,
               wfex, wfeg, bfe,
               wh0, bh0, wh1, bh1, wfp, wfx, bf,
               pc_o, h0_o, styleg_o, a_o, c_o):
    x2 = x2_r[...]
    z = z_r[...]
    b = batch_r[...]
    gg = _leaky(gmax_r[...] @ wgc0[...] + bgc0[...], 0.01)
    gg = _leaky(gg @ wgc1[...] + bgc1[...], 0.01)
    gsel = jnp.where(b == 0, gg[0:1, :], gg[1:2, :])
    t = _leaky(x2 @ wt0x[...] + gsel @ wt0g[...] + bt0[...], 0.01)
    t = _leaky(t @ wt1[...] + bt1[...], 0.01)
    pc = jnp.tanh(t @ wt2[...] + bt2[...]) * 0.75
    pc_o[...] = pc
    sg = _leaky(z @ w0zg[...] + pc @ w0pg[...] + b0g[...], 0.01)
    styleg = _leaky(sg @ w1g[...] + b1g[...], 0.01)
    styleg_o[...] = styleg
    h0 = _leaky(x2 @ wfex[...] + gsel @ wfeg[...] + bfe[...], 0.01)
    h0_o[...] = h0
    a, c = _hdelta_ac(h0, pc, wh0[...], bh0[...], wh1[...], bh1[...],
                      wfp[...], wfx[...], bf[...])
    a_o[...] = a
    c_o[...] = c


def _tc4_body(h0_r, p0_r, p1_r, styleg_r,
              wg0, bg0, wg1, bg1, wag, bag, wab, bab,
              hout_o):
    hout_o[...] = _ada_update(h0_r[...], p0_r[...], p1_r[...], styleg_r[...],
                              wg0[...], bg0[...], wg1[...], bg1[...],
                              wag[...], bag[...], wab[...], bab[...])


def _wspecs(ws):
    return [_full_spec(w.shape) for w in ws]


def _blk_weights(bp):
    wh0 = bp['h0'][0].T
    bh0 = bp['h0'][1][None, :]
    wh1 = bp['h1'][0].T
    bh1 = bp['h1'][1][None, :]
    wf = bp['f0'][0]
    wfp = wf[:, :3].T
    wfx = wf[:, 3:].T
    bf = bp['f0'][1][None, :]
    wg0 = bp['g0'][0].T
    bg0 = bp['g0'][1][None, :]
    wg1 = bp['g1'][0].T
    bg1 = bp['g1'][1][None, :]
    wa = bp['ada'][0]
    ba = bp['ada'][1]
    wag = wa[:CHN].T
    bag = ba[None, :CHN]
    wab = wa[CHN:].T
    bab = ba[None, CHN:]
    return dict(wh0=wh0, bh0=bh0, wh1=wh1, bh1=bh1, wfp=wfp, wfx=wfx, bf=bf,
                wg0=wg0, bg0=bg0, wg1=wg1, bg1=bg1,
                wag=wag, bag=bag, wab=wab, bab=bab)


def kernel(pos, edge_index, batch, z, params):
    cp = params['cloud']
    gp = params['gauss']
    src = edge_index[0]
    dst = edge_index[1]
    batch2 = batch[:, None]

    w0 = cp['style0'][0]
    w0z, w0p = w0[:, :128].T, w0[:, 128:].T
    b0 = cp['style0'][1][None, :]
    w1 = cp['style1'][0].T
    b1 = cp['style1'][1][None, :]
    br = cp['Brff']
    brcat = jnp.concatenate([br.T, br.T], axis=1)
    phase = jnp.concatenate([jnp.zeros((1, 64), jnp.float32),
                             jnp.full((1, 64), 0.5 * jnp.pi, jnp.float32)],
                            axis=1)
    cb0 = _blk_weights(cp['blocks'][0])
    cb1 = _blk_weights(cp['blocks'][1])
    gb0 = _blk_weights(gp['blocks'][0])
    wgc0 = cp['gc0'][0].T
    bgc0 = cp['gc0'][1][None, :]
    wgc1 = cp['gc1'][0].T
    bgc1 = cp['gc1'][1][None, :]
    wt0 = cp['tail0'][0]
    wt0x, wt0g = wt0[:, :CHN].T, wt0[:, CHN:].T
    bt0 = cp['tail0'][1][None, :]
    wt1 = cp['tail1'][0].T
    bt1 = cp['tail1'][1][None, :]
    wt2 = cp['tail2'][0].T
    bt2 = cp['tail2'][1][None, :]
    wg0s = gp['style0'][0]
    w0zg, w0pg = wg0s[:, :128].T, wg0s[:, 128:].T
    b0g = gp['style0'][1][None, :]
    w1g = gp['style1'][0].T
    b1g = gp['style1'][1][None, :]
    wfe = gp['fe'][0]
    wfex, wfeg = wfe[:, :CHN].T, wfe[:, CHN:].T
    bfe = gp['fe'][1][None, :]

    f32 = jnp.float32
    rows = lambda c: jax.ShapeDtypeStruct((N, c), f32)

    ws1 = [w0z, w0p, b0, w1, b1, brcat, phase,
           cb0['wh0'], cb0['bh0'], cb0['wh1'], cb0['bh1'],
           cb0['wfp'], cb0['wfx'], cb0['bf']]
    style, x0, a1, c1 = pl.pallas_call(
        _tc1_body,
        grid=(GRID,),
        in_specs=[_row_spec(3), _row_spec(CHN)] + _wspecs(ws1),
        out_specs=[_row_spec(CHN)] * 4,
        out_shape=[rows(CHN)] * 4,
    )(pos, z, *ws1)

    pt1a, pt1b = _edge_agg(a1, c1, src, dst)

    ws2 = [cb0['wg0'], cb0['bg0'], cb0['wg1'], cb0['bg1'],
           cb0['wag'], cb0['bag'], cb0['wab'], cb0['bab'],
           cb1['wh0'], cb1['bh0'], cb1['wh1'], cb1['bh1'],
           cb1['wfp'], cb1['wfx'], cb1['bf']]
    x1, a2, c2 = pl.pallas_call(
        _tc2_body,
        grid=(GRID,),
        in_specs=[_row_spec(CHN)] * 4 + [_row_spec(3)] + _wspecs(ws2),
        out_specs=[_row_spec(CHN)] * 3,
        out_shape=[rows(CHN)] * 3,
    )(x0, pt1a, pt1b, style, pos, *ws2)

    pt2a, pt2b = _edge_agg(a2, c2, src, dst)

    ws3a = [cb1['wg0'], cb1['bg0'], cb1['wg1'], cb1['bg1'],
            cb1['wag'], cb1['bag'], cb1['wab'], cb1['bab']]
    x2, gmax = pl.pallas_call(
        _tc3a_body,
        grid=(GRID,),
        in_specs=[_row_spec(CHN)] * 4 + [_row_spec(1)] + _wspecs(ws3a),
        out_specs=[_row_spec(CHN), _full_spec((2, CHN))],
        out_shape=[rows(CHN), jax.ShapeDtypeStruct((2, CHN), f32)],
    )(x1, pt2a, pt2b, style, batch2, *ws3a)

    ws3b = [wgc0, bgc0, wgc1, bgc1,
            wt0x, wt0g, bt0, wt1, bt1, wt2, bt2,
            w0zg, w0pg, b0g, w1g, b1g,
            wfex, wfeg, bfe,
            gb0['wh0'], gb0['bh0'], gb0['wh1'], gb0['bh1'],
            gb0['wfp'], gb0['wfx'], gb0['bf']]
    pc, h0, styleg, a3, c3 = pl.pallas_call(
        _tc3b_body,
        grid=(GRID,),
        in_specs=[_row_spec(CHN), _row_spec(CHN), _row_spec(1),
                  _full_spec((2, CHN))] + _wspecs(ws3b),
        out_specs=[_row_spec(3)] + [_row_spec(CHN)] * 4,
        out_shape=[rows(3)] + [rows(CHN)] * 4,
    )(x2, z, batch2, gmax, *ws3b)

    pt3a, pt3b = _edge_agg(a3, c3, src, dst)

    ws4 = [gb0['wg0'], gb0['bg0'], gb0['wg1'], gb0['bg1'],
           gb0['wag'], gb0['bag'], gb0['wab'], gb0['bab']]
    hout = pl.pallas_call(
        _tc4_body,
        grid=(GRID,),
        in_specs=[_row_spec(CHN)] * 4 + _wspecs(ws4),
        out_specs=_row_spec(CHN),
        out_shape=rows(CHN),
    )(h0, pt3a, pt3b, styleg, *ws4)

    return jnp.concatenate([pc, hout], axis=-1)


# speedup vs baseline: 10.1023x; 2.0771x over previous
import functools

import jax
import jax.numpy as jnp
from jax import lax
from jax.experimental import pallas as pl
from jax.experimental.pallas import tpu as pltpu
from jax.experimental.pallas import tpu_sc as plsc

N = 10000
E = 320000
CHN = 128
EPS = 1e-5
BLK = 2000
GRID = N // BLK

_NC, _NS = 2, 16
_NW = _NC * _NS
_EPT = E // _NW
_K = 80
_NCH = _EPT // _K
_NPAD = 10240
_RPT = _NPAD // _NS


def _leaky(x, s):
    return jnp.where(x > 0, x, s * x)


_NBUF = 2
_NIDX = 4


def _edge_body(a_hbm, c_hbm, src_hbm, dst_hbm, out_hbm,
               si0, si1, si2, si3, di0, di1, di2, di3,
               av0, av1, cv0, cv1, acc,
               is0, is1, is2, is3, sa0, sa1, sc0, sc1):
    sis = [si0, si1, si2, si3]
    dis = [di0, di1, di2, di3]
    iss = [is0, is1, is2, is3]
    avs = [av0, av1]
    cvs = [cv0, cv1]
    sas = [sa0, sa1]
    scs = [sc0, sc1]
    cid = lax.axis_index("c")
    sid = lax.axis_index("s")
    wid = cid * _NS + sid

    def idx_load(chunk, q):
        base = wid * _EPT + chunk * _K
        pltpu.sync_copy(src_hbm.at[pl.ds(base, _K)], sis[q])
        pltpu.sync_copy(dst_hbm.at[pl.ds(base, _K)], dis[q])

    def zrow(j, _):
        e = j // 8
        d = (j % 8) * 16
        av0[e, pl.ds(d, 16)] = jnp.zeros((16,), jnp.float32)
        return 0
    lax.fori_loop(0, _K * 8, zrow, 0)

    def zcp(j, _):
        pltpu.sync_copy(av0, acc.at[pl.ds(sid * _RPT + j * _K, _K)])
        return 0
    lax.fori_loop(0, _RPT // _K, zcp, 0)

    for b in range(_NBUF):
        idx_load(b, b)
        pltpu.async_copy(a_hbm.at[sis[b]], avs[b], sas[b])
        pltpu.async_copy(c_hbm.at[dis[b]], cvs[b], scs[b])
    plsc.subcore_barrier()

    def slot(i, k):
        b = k % _NBUF
        q = k
        q2 = (k + _NBUF) % _NIDX
        pltpu.make_async_copy(a_hbm.at[sis[q]], avs[b], sas[b]).wait()
        pltpu.make_async_copy(c_hbm.at[dis[q]], cvs[b], scs[b]).wait()

        def ebody(e, _):
            for dd in range(8):
                d = dd * 16
                a = avs[b][e, pl.ds(d, 16)]
                c = cvs[b][e, pl.ds(d, 16)]
                cvs[b][e, pl.ds(d, 16)] = jnp.maximum(a + c, 0.0)
            return 0
        lax.fori_loop(0, _K, ebody, 0)

        pltpu.sync_copy(cvs[b], acc.at[dis[q]], add=True)

        @pl.when(i + _NBUF < _NCH)
        def _():
            idx_load(i + _NBUF, q2)
            pltpu.async_copy(a_hbm.at[sis[q2]], avs[b], sas[b])
            pltpu.async_copy(c_hbm.at[dis[q2]], cvs[b], scs[b])


    def outer(g, _):
        for k in range(_NIDX):
            i = g * _NIDX + k

            @pl.when(i < _NCH)
            def _():
                slot(i, k)
        return 0
    lax.fori_loop(0, (_NCH + _NIDX - 1) // _NIDX, outer, 0)
    plsc.subcore_barrier()
    pltpu.sync_copy(acc.at[pl.ds(sid * _RPT, _RPT)],
                    out_hbm.at[cid, pl.ds(sid * _RPT, _RPT)])


@functools.cache
def _edge_agg_fn():
    return pl.kernel(
        _edge_body,
        out_type=jax.ShapeDtypeStruct((2, _NPAD, CHN), jnp.float32),
        mesh=plsc.VectorSubcoreMesh(core_axis_name="c", subcore_axis_name="s",
                                    num_cores=_NC, num_subcores=_NS),
        scratch_types=(
            [pltpu.VMEM((_K,), jnp.int32)] * (2 * _NIDX)
            + [pltpu.VMEM((_K, CHN), jnp.float32)] * (2 * _NBUF)
            + [pltpu.VMEM_SHARED((_NPAD, CHN), jnp.float32)]
            + [pltpu.SemaphoreType.DMA] * (_NIDX + 2 * _NBUF)
        ),
    )


def _edge_agg(a, c, src, dst):
    parts = _edge_agg_fn()(a, c, src, dst)
    return parts[0, :N], parts[1, :N]


def _row_spec(cols):
    return pl.BlockSpec((BLK, cols), lambda i: (i, 0))


def _full_spec(shape):
    return pl.BlockSpec(shape, lambda i: (0,) * len(shape))


def _ada_update(x, p0, p1, style, wg0, bg0, wg1, bg1, wag, bag, wab, bab):
    agg = p0 + p1
    t = jnp.maximum(agg @ wg0 + bg0, 0.0)
    o = x + t @ wg1 + bg1
    o = _leaky(o, 0.2)
    gamma = style @ wag + bag
    beta = style @ wab + bab
    mu = jnp.mean(o, axis=1, keepdims=True)
    var = jnp.mean((o - mu) ** 2, axis=1, keepdims=True)
    xn = (o - mu) * lax.rsqrt(var + EPS)
    return gamma * xn + beta


def _hdelta_ac(x, pos, wh0, bh0, wh1, bh1, wfp, wfx, bf):
    h = jnp.maximum(x @ wh0 + bh0, 0.0)
    delta = jnp.tanh(h @ wh1 + bh1)
    a = pos @ wfp + x @ wfx
    c = (delta - pos) @ wfp + bf
    return a, c


def _tc1_body(pos_r, z_r, w0z, w0p, b0, w1, b1, brcat, phase,
              wh0, bh0, wh1, bh1, wfp, wfx, bf,
              style_o, x0_o, a_o, c_o):
    pos = pos_r[...]
    z = z_r[...]
    s = _leaky(z @ w0z[...] + pos @ w0p[...] + b0[...], 0.01)
    style = _leaky(s @ w1[...] + b1[...], 0.01)
    style_o[...] = style
    v = ((2.0 * jnp.pi) * pos) @ brcat[...] - phase[...]
    c1 = jnp.float32(6.2831855)
    c2 = jnp.float32(-1.7484555e-07)
    k = jnp.round(v * jnp.float32(1.0 / (2.0 * jnp.pi)))
    r = (v - k * c1) - k * c2
    x0 = jnp.cos(r)
    x0_o[...] = x0
    a, c = _hdelta_ac(x0, pos, wh0[...], bh0[...], wh1[...], bh1[...],
                      wfp[...], wfx[...], bf[...])
    a_o[...] = a
    c_o[...] = c


def _tc2_body(x_r, p0_r, p1_r, style_r, pos_r,
              wg0, bg0, wg1, bg1, wag, bag, wab, bab,
              wh0, bh0, wh1, bh1, wfp, wfx, bf,
              x1_o, a_o, c_o):
    x1 = _ada_update(x_r[...], p0_r[...], p1_r[...], style_r[...],
                     wg0[...], bg0[...], wg1[...], bg1[...],
                     wag[...], bag[...], wab[...], bab[...])
    x1_o[...] = x1
    a, c = _hdelta_ac(x1, pos_r[...], wh0[...], bh0[...], wh1[...], bh1[...],
                      wfp[...], wfx[...], bf[...])
    a_o[...] = a
    c_o[...] = c


def _tc3a_body(x_r, p0_r, p1_r, style_r, batch_r,
               wg0, bg0, wg1, bg1, wag, bag, wab, bab,
               x2_o, gmax_o):
    x2 = _ada_update(x_r[...], p0_r[...], p1_r[...], style_r[...],
                     wg0[...], bg0[...], wg1[...], bg1[...],
                     wag[...], bag[...], wab[...], bab[...])
    x2_o[...] = x2
    b = batch_r[...]
    neg = jnp.full_like(x2, -jnp.inf)
    g0 = jnp.max(jnp.where(b == 0, x2, neg), axis=0, keepdims=True)
    g1 = jnp.max(jnp.where(b == 1, x2, neg), axis=0, keepdims=True)
    cur = jnp.concatenate([g0, g1], axis=0)
    i = pl.program_id(0)

    @pl.when(i == 0)
    def _():
        gmax_o[...] = cur

    @pl.when(i != 0)
    def _():
        gmax_o[...] = jnp.maximum(gmax_o[...], cur)


def _tc3b_body(x2_r, z_r, batch_r, gmax_r,
               wgc0, bgc0, wgc1, bgc1,
               wt0x, wt0g, bt0, wt1, bt1, wt2, bt2,
               w0zg, w0pg, b0g, w1g, b1g,
               wfex, wfeg, bfe,
               wh0, bh0, wh1, bh1, wfp, wfx, bf,
               pc_o, h0_o, styleg_o, a_o, c_o):
    x2 = x2_r[...]
    z = z_r[...]
    b = batch_r[...]
    gg = _leaky(gmax_r[...] @ wgc0[...] + bgc0[...], 0.01)
    gg = _leaky(gg @ wgc1[...] + bgc1[...], 0.01)
    gsel = jnp.where(b == 0, gg[0:1, :], gg[1:2, :])
    t = _leaky(x2 @ wt0x[...] + gsel @ wt0g[...] + bt0[...], 0.01)
    t = _leaky(t @ wt1[...] + bt1[...], 0.01)
    pc = jnp.tanh(t @ wt2[...] + bt2[...]) * 0.75
    pc_o[...] = pc
    sg = _leaky(z @ w0zg[...] + pc @ w0pg[...] + b0g[...], 0.01)
    styleg = _leaky(sg @ w1g[...] + b1g[...], 0.01)
    styleg_o[...] = styleg
    h0 = _leaky(x2 @ wfex[...] + gsel @ wfeg[...] + bfe[...], 0.01)
    h0_o[...] = h0
    a, c = _hdelta_ac(h0, pc, wh0[...], bh0[...], wh1[...], bh1[...],
                      wfp[...], wfx[...], bf[...])
    a_o[...] = a
    c_o[...] = c


def _tc4_body(h0_r, p0_r, p1_r, styleg_r,
              wg0, bg0, wg1, bg1, wag, bag, wab, bab,
              hout_o):
    hout_o[...] = _ada_update(h0_r[...], p0_r[...], p1_r[...], styleg_r[...],
                              wg0[...], bg0[...], wg1[...], bg1[...],
                              wag[...], bag[...], wab[...], bab[...])


def _wspecs(ws):
    return [_full_spec(w.shape) for w in ws]


def _blk_weights(bp):
    wh0 = bp['h0'][0].T
    bh0 = bp['h0'][1][None, :]
    wh1 = bp['h1'][0].T
    bh1 = bp['h1'][1][None, :]
    wf = bp['f0'][0]
    wfp = wf[:, :3].T
    wfx = wf[:, 3:].T
    bf = bp['f0'][1][None, :]
    wg0 = bp['g0'][0].T
    bg0 = bp['g0'][1][None, :]
    wg1 = bp['g1'][0].T
    bg1 = bp['g1'][1][None, :]
    wa = bp['ada'][0]
    ba = bp['ada'][1]
    wag = wa[:CHN].T
    bag = ba[None, :CHN]
    wab = wa[CHN:].T
    bab = ba[None, CHN:]
    return dict(wh0=wh0, bh0=bh0, wh1=wh1, bh1=bh1, wfp=wfp, wfx=wfx, bf=bf,
                wg0=wg0, bg0=bg0, wg1=wg1, bg1=bg1,
                wag=wag, bag=bag, wab=wab, bab=bab)


def kernel(pos, edge_index, batch, z, params):
    cp = params['cloud']
    gp = params['gauss']
    src = edge_index[0]
    dst = edge_index[1]
    batch2 = batch[:, None]

    w0 = cp['style0'][0]
    w0z, w0p = w0[:, :128].T, w0[:, 128:].T
    b0 = cp['style0'][1][None, :]
    w1 = cp['style1'][0].T
    b1 = cp['style1'][1][None, :]
    br = cp['Brff']
    brcat = jnp.concatenate([br.T, br.T], axis=1)
    phase = jnp.concatenate([jnp.zeros((1, 64), jnp.float32),
                             jnp.full((1, 64), 0.5 * jnp.pi, jnp.float32)],
                            axis=1)
    cb0 = _blk_weights(cp['blocks'][0])
    cb1 = _blk_weights(cp['blocks'][1])
    gb0 = _blk_weights(gp['blocks'][0])
    wgc0 = cp['gc0'][0].T
    bgc0 = cp['gc0'][1][None, :]
    wgc1 = cp['gc1'][0].T
    bgc1 = cp['gc1'][1][None, :]
    wt0 = cp['tail0'][0]
    wt0x, wt0g = wt0[:, :CHN].T, wt0[:, CHN:].T
    bt0 = cp['tail0'][1][None, :]
    wt1 = cp['tail1'][0].T
    bt1 = cp['tail1'][1][None, :]
    wt2 = cp['tail2'][0].T
    bt2 = cp['tail2'][1][None, :]
    wg0s = gp['style0'][0]
    w0zg, w0pg = wg0s[:, :128].T, wg0s[:, 128:].T
    b0g = gp['style0'][1][None, :]
    w1g = gp['style1'][0].T
    b1g = gp['style1'][1][None, :]
    wfe = gp['fe'][0]
    wfex, wfeg = wfe[:, :CHN].T, wfe[:, CHN:].T
    bfe = gp['fe'][1][None, :]

    f32 = jnp.float32
    rows = lambda c: jax.ShapeDtypeStruct((N, c), f32)

    ws1 = [w0z, w0p, b0, w1, b1, brcat, phase,
           cb0['wh0'], cb0['bh0'], cb0['wh1'], cb0['bh1'],
           cb0['wfp'], cb0['wfx'], cb0['bf']]
    style, x0, a1, c1 = pl.pallas_call(
        _tc1_body,
        grid=(GRID,),
        in_specs=[_row_spec(3), _row_spec(CHN)] + _wspecs(ws1),
        out_specs=[_row_spec(CHN)] * 4,
        out_shape=[rows(CHN)] * 4,
    )(pos, z, *ws1)

    pt1a, pt1b = _edge_agg(a1, c1, src, dst)

    ws2 = [cb0['wg0'], cb0['bg0'], cb0['wg1'], cb0['bg1'],
           cb0['wag'], cb0['bag'], cb0['wab'], cb0['bab'],
           cb1['wh0'], cb1['bh0'], cb1['wh1'], cb1['bh1'],
           cb1['wfp'], cb1['wfx'], cb1['bf']]
    x1, a2, c2 = pl.pallas_call(
        _tc2_body,
        grid=(GRID,),
        in_specs=[_row_spec(CHN)] * 4 + [_row_spec(3)] + _wspecs(ws2),
        out_specs=[_row_spec(CHN)] * 3,
        out_shape=[rows(CHN)] * 3,
    )(x0, pt1a, pt1b, style, pos, *ws2)

    pt2a, pt2b = _edge_agg(a2, c2, src, dst)

    ws3a = [cb1['wg0'], cb1['bg0'], cb1['wg1'], cb1['bg1'],
            cb1['wag'], cb1['bag'], cb1['wab'], cb1['bab']]
    x2, gmax = pl.pallas_call(
        _tc3a_body,
        grid=(GRID,),
        in_specs=[_row_spec(CHN)] * 4 + [_row_spec(1)] + _wspecs(ws3a),
        out_specs=[_row_spec(CHN), _full_spec((2, CHN))],
        out_shape=[rows(CHN), jax.ShapeDtypeStruct((2, CHN), f32)],
    )(x1, pt2a, pt2b, style, batch2, *ws3a)

    ws3b = [wgc0, bgc0, wgc1, bgc1,
            wt0x, wt0g, bt0, wt1, bt1, wt2, bt2,
            w0zg, w0pg, b0g, w1g, b1g,
            wfex, wfeg, bfe,
            gb0['wh0'], gb0['bh0'], gb0['wh1'], gb0['bh1'],
            gb0['wfp'], gb0['wfx'], gb0['bf']]
    pc, h0, styleg, a3, c3 = pl.pallas_call(
        _tc3b_body,
        grid=(GRID,),
        in_specs=[_row_spec(CHN), _row_spec(CHN), _row_spec(1),
                  _full_spec((2, CHN))] + _wspecs(ws3b),
        out_specs=[_row_spec(3)] + [_row_spec(CHN)] * 4,
        out_shape=[rows(3)] + [rows(CHN)] * 4,
    )(x2, z, batch2, gmax, *ws3b)

    pt3a, pt3b = _edge_agg(a3, c3, src, dst)

    ws4 = [gb0['wg0'], gb0['bg0'], gb0['wg1'], gb0['bg1'],
           gb0['wag'], gb0['bag'], gb0['wab'], gb0['bab']]
    hout = pl.pallas_call(
        _tc4_body,
        grid=(GRID,),
        in_specs=[_row_spec(CHN)] * 4 + _wspecs(ws4),
        out_specs=_row_spec(CHN),
        out_shape=rows(CHN),
    )(h0, pt3a, pt3b, styleg, *ws4)

    return jnp.concatenate([pc, hout], axis=-1)


# speedup vs baseline: 13.4848x; 1.3348x over previous
import functools

import jax
import jax.numpy as jnp
from jax import lax
from jax.experimental import pallas as pl
from jax.experimental.pallas import tpu as pltpu
from jax.experimental.pallas import tpu_sc as plsc

N = 10000
E = 320000
CHN = 128
EPS = 1e-5
BLK = 2000
GRID = N // BLK

_NC, _NS = 2, 16
_NW = _NC * _NS
_EPT = E // _NW
_K = 80
_NCH = _EPT // _K
_NPAD = 10240
_RPT = _NPAD // _NS


def _leaky(x, s):
    return jnp.where(x > 0, x, s * x)


_NBUF = 2
_NIDX = 4


def _edge_body(a_hbm, c_hbm, src_hbm, dst_hbm, out_hbm,
               si0, si1, si2, si3, di0, di1, di2, di3,
               av0, av1, cv0, cv1, acc,
               is0, is1, is2, is3, sa0, sa1, sc0, sc1):
    sis = [si0, si1, si2, si3]
    dis = [di0, di1, di2, di3]
    iss = [is0, is1, is2, is3]
    avs = [av0, av1]
    cvs = [cv0, cv1]
    sas = [sa0, sa1]
    scs = [sc0, sc1]
    cid = lax.axis_index("c")
    sid = lax.axis_index("s")
    wid = cid * _NS + sid

    def idx_load(chunk, q):
        base = wid * _EPT + chunk * _K
        pltpu.sync_copy(src_hbm.at[pl.ds(base, _K)], sis[q])
        pltpu.sync_copy(dst_hbm.at[pl.ds(base, _K)], dis[q])

    def idx_start(chunk, q):
        base = wid * _EPT + chunk * _K
        pltpu.async_copy(src_hbm.at[pl.ds(base, _K)], sis[q], iss[q])
        pltpu.async_copy(dst_hbm.at[pl.ds(base, _K)], dis[q], iss[q])

    def idx_wait(chunk, q):
        base = wid * _EPT + chunk * _K
        pltpu.make_async_copy(src_hbm.at[pl.ds(base, _K)], sis[q],
                              iss[q]).wait()
        pltpu.make_async_copy(dst_hbm.at[pl.ds(base, _K)], dis[q],
                              iss[q]).wait()

    def zrow(j, _):
        e = j // 8
        d = (j % 8) * 16
        av0[e, pl.ds(d, 16)] = jnp.zeros((16,), jnp.float32)
        return 0
    lax.fori_loop(0, _K * 8, zrow, 0)

    def zcp(j, _):
        pltpu.sync_copy(av0, acc.at[pl.ds(sid * _RPT + j * _K, _K)])
        return 0
    lax.fori_loop(0, _RPT // _K, zcp, 0)

    for b in range(_NBUF):
        idx_load(b, b)
        pltpu.async_copy(a_hbm.at[sis[b]], avs[b], sas[b])
        pltpu.async_copy(c_hbm.at[dis[b]], cvs[b], scs[b])
    plsc.subcore_barrier()

    def slot(i, k):
        b = k % _NBUF
        q = k
        q2 = (k + _NBUF) % _NIDX
        pltpu.make_async_copy(a_hbm.at[sis[q]], avs[b], sas[b]).wait()
        pltpu.make_async_copy(c_hbm.at[dis[q]], cvs[b], scs[b]).wait()

        @pl.when(i + _NBUF < _NCH)
        def _():
            idx_start(i + _NBUF, q2)

        def ebody(e2, _):
            for ee in range(2):
                e = e2 * 2 + ee
                for dd in range(8):
                    d = dd * 16
                    a = avs[b][e, pl.ds(d, 16)]
                    c = cvs[b][e, pl.ds(d, 16)]
                    cvs[b][e, pl.ds(d, 16)] = jnp.maximum(a + c, 0.0)
            return 0
        lax.fori_loop(0, _K // 2, ebody, 0)

        pltpu.sync_copy(cvs[b], acc.at[dis[q]], add=True)

        @pl.when(i + _NBUF < _NCH)
        def _():
            idx_wait(i + _NBUF, q2)
            pltpu.async_copy(a_hbm.at[sis[q2]], avs[b], sas[b])
            pltpu.async_copy(c_hbm.at[dis[q2]], cvs[b], scs[b])


    def outer(g, _):
        for k in range(_NIDX):
            i = g * _NIDX + k

            @pl.when(i < _NCH)
            def _():
                slot(i, k)
        return 0
    lax.fori_loop(0, (_NCH + _NIDX - 1) // _NIDX, outer, 0)
    plsc.subcore_barrier()
    pltpu.sync_copy(acc.at[pl.ds(sid * _RPT, _RPT)],
                    out_hbm.at[cid, pl.ds(sid * _RPT, _RPT)])


@functools.cache
def _edge_agg_fn():
    return pl.kernel(
        _edge_body,
        out_type=jax.ShapeDtypeStruct((2, _NPAD, CHN), jnp.float32),
        mesh=plsc.VectorSubcoreMesh(core_axis_name="c", subcore_axis_name="s",
                                    num_cores=_NC, num_subcores=_NS),
        scratch_types=(
            [pltpu.VMEM((_K,), jnp.int32)] * (2 * _NIDX)
            + [pltpu.VMEM((_K, CHN), jnp.float32)] * (2 * _NBUF)
            + [pltpu.VMEM_SHARED((_NPAD, CHN), jnp.float32)]
            + [pltpu.SemaphoreType.DMA] * (_NIDX + 2 * _NBUF)
        ),
    )


def _edge_agg(a, c, src, dst):
    parts = _edge_agg_fn()(a, c, src, dst)
    return parts[0, :N], parts[1, :N]


def _row_spec(cols):
    return pl.BlockSpec((BLK, cols), lambda i: (i, 0))


def _full_spec(shape):
    return pl.BlockSpec(shape, lambda i: (0,) * len(shape))


def _ada_update(x, p0, p1, style, wg0, bg0, wg1, bg1, wag, bag, wab, bab):
    agg = p0 + p1
    t = jnp.maximum(agg @ wg0 + bg0, 0.0)
    o = x + t @ wg1 + bg1
    o = _leaky(o, 0.2)
    gamma = style @ wag + bag
    beta = style @ wab + bab
    mu = jnp.mean(o, axis=1, keepdims=True)
    var = jnp.mean((o - mu) ** 2, axis=1, keepdims=True)
    xn = (o - mu) * lax.rsqrt(var + EPS)
    return gamma * xn + beta


def _hdelta_ac(x, pos, wh0, bh0, wh1, bh1, wfp, wfx, bf):
    h = jnp.maximum(x @ wh0 + bh0, 0.0)
    delta = jnp.tanh(h @ wh1 + bh1)
    a = pos @ wfp + x @ wfx
    c = (delta - pos) @ wfp + bf
    return a, c


def _tc1_body(pos_r, z_r, w0z, w0p, b0, w1, b1, brcat, phase,
              wh0, bh0, wh1, bh1, wfp, wfx, bf,
              style_o, x0_o, a_o, c_o):
    pos = pos_r[...]
    z = z_r[...]
    s = _leaky(z @ w0z[...] + pos @ w0p[...] + b0[...], 0.01)
    style = _leaky(s @ w1[...] + b1[...], 0.01)
    style_o[...] = style
    v = ((2.0 * jnp.pi) * pos) @ brcat[...] - phase[...]
    c1 = jnp.float32(6.2831855)
    c2 = jnp.float32(-1.7484555e-07)
    k = jnp.round(v * jnp.float32(1.0 / (2.0 * jnp.pi)))
    r = (v - k * c1) - k * c2
    x0 = jnp.cos(r)
    x0_o[...] = x0
    a, c = _hdelta_ac(x0, pos, wh0[...], bh0[...], wh1[...], bh1[...],
                      wfp[...], wfx[...], bf[...])
    a_o[...] = a
    c_o[...] = c


def _tc2_body(x_r, p0_r, p1_r, style_r, pos_r,
              wg0, bg0, wg1, bg1, wag, bag, wab, bab,
              wh0, bh0, wh1, bh1, wfp, wfx, bf,
              x1_o, a_o, c_o):
    x1 = _ada_update(x_r[...], p0_r[...], p1_r[...], style_r[...],
                     wg0[...], bg0[...], wg1[...], bg1[...],
                     wag[...], bag[...], wab[...], bab[...])
    x1_o[...] = x1
    a, c = _hdelta_ac(x1, pos_r[...], wh0[...], bh0[...], wh1[...], bh1[...],
                      wfp[...], wfx[...], bf[...])
    a_o[...] = a
    c_o[...] = c


def _tc3a_body(x_r, p0_r, p1_r, style_r, batch_r,
               wg0, bg0, wg1, bg1, wag, bag, wab, bab,
               x2_o, gmax_o):
    x2 = _ada_update(x_r[...], p0_r[...], p1_r[...], style_r[...],
                     wg0[...], bg0[...], wg1[...], bg1[...],
                     wag[...], bag[...], wab[...], bab[...])
    x2_o[...] = x2
    b = batch_r[...]
    neg = jnp.full_like(x2, -jnp.inf)
    g0 = jnp.max(jnp.where(b == 0, x2, neg), axis=0, keepdims=True)
    g1 = jnp.max(jnp.where(b == 1, x2, neg), axis=0, keepdims=True)
    cur = jnp.concatenate([g0, g1], axis=0)
    i = pl.program_id(0)

    @pl.when(i == 0)
    def _():
        gmax_o[...] = cur

    @pl.when(i != 0)
    def _():
        gmax_o[...] = jnp.maximum(gmax_o[...], cur)


def _tc3b_body(x2_r, z_r, batch_r, gmax_r,
               wgc0, bgc0, wgc1, bgc1,
               wt0x, wt0g, bt0, wt1, bt1, wt2, bt2,
               w0zg, w0pg, b0g, w1g, b1g,
               wfex, wfeg, bfe,
               wh0, bh0, wh1, bh1, wfp, wfx, bf,
               pc_o, h0_o, styleg_o, a_o, c_o):
    x2 = x2_r[...]
    z = z_r[...]
    b = batch_r[...]
    gg = _leaky(gmax_r[...] @ wgc0[...] + bgc0[...], 0.01)
    gg = _leaky(gg @ wgc1[...] + bgc1[...], 0.01)
    gsel = jnp.where(b == 0, gg[0:1, :], gg[1:2, :])
    t = _leaky(x2 @ wt0x[...] + gsel @ wt0g[...] + bt0[...], 0.01)
    t = _leaky(t @ wt1[...] + bt1[...], 0.01)
    pc = jnp.tanh(t @ wt2[...] + bt2[...]) * 0.75
    pc_o[...] = pc
    sg = _leaky(z @ w0zg[...] + pc @ w0pg[...] + b0g[...], 0.01)
    styleg = _leaky(sg @ w1g[...] + b1g[...], 0.01)
    styleg_o[...] = styleg
    h0 = _leaky(x2 @ wfex[...] + gsel @ wfeg[...] + bfe[...], 0.01)
    h0_o[...] = h0
    a, c = _hdelta_ac(h0, pc, wh0[...], bh0[...], wh1[...], bh1[...],
                      wfp[...], wfx[...], bf[...])
    a_o[...] = a
    c_o[...] = c


def _tc4_body(h0_r, p0_r, p1_r, styleg_r,
              wg0, bg0, wg1, bg1, wag, bag, wab, bab,
              hout_o):
    hout_o[...] = _ada_update(h0_r[...], p0_r[...], p1_r[...], styleg_r[...],
                              wg0[...], bg0[...], wg1[...], bg1[...],
                              wag[...], bag[...], wab[...], bab[...])


def _wspecs(ws):
    return [_full_spec(w.shape) for w in ws]


def _blk_weights(bp):
    wh0 = bp['h0'][0].T
    bh0 = bp['h0'][1][None, :]
    wh1 = bp['h1'][0].T
    bh1 = bp['h1'][1][None, :]
    wf = bp['f0'][0]
    wfp = wf[:, :3].T
    wfx = wf[:, 3:].T
    bf = bp['f0'][1][None, :]
    wg0 = bp['g0'][0].T
    bg0 = bp['g0'][1][None, :]
    wg1 = bp['g1'][0].T
    bg1 = bp['g1'][1][None, :]
    wa = bp['ada'][0]
    ba = bp['ada'][1]
    wag = wa[:CHN].T
    bag = ba[None, :CHN]
    wab = wa[CHN:].T
    bab = ba[None, CHN:]
    return dict(wh0=wh0, bh0=bh0, wh1=wh1, bh1=bh1, wfp=wfp, wfx=wfx, bf=bf,
                wg0=wg0, bg0=bg0, wg1=wg1, bg1=bg1,
                wag=wag, bag=bag, wab=wab, bab=bab)


def kernel(pos, edge_index, batch, z, params):
    cp = params['cloud']
    gp = params['gauss']
    src = edge_index[0]
    dst = edge_index[1]
    batch2 = batch[:, None]

    w0 = cp['style0'][0]
    w0z, w0p = w0[:, :128].T, w0[:, 128:].T
    b0 = cp['style0'][1][None, :]
    w1 = cp['style1'][0].T
    b1 = cp['style1'][1][None, :]
    br = cp['Brff']
    brcat = jnp.concatenate([br.T, br.T], axis=1)
    phase = jnp.concatenate([jnp.zeros((1, 64), jnp.float32),
                             jnp.full((1, 64), 0.5 * jnp.pi, jnp.float32)],
                            axis=1)
    cb0 = _blk_weights(cp['blocks'][0])
    cb1 = _blk_weights(cp['blocks'][1])
    gb0 = _blk_weights(gp['blocks'][0])
    wgc0 = cp['gc0'][0].T
    bgc0 = cp['gc0'][1][None, :]
    wgc1 = cp['gc1'][0].T
    bgc1 = cp['gc1'][1][None, :]
    wt0 = cp['tail0'][0]
    wt0x, wt0g = wt0[:, :CHN].T, wt0[:, CHN:].T
    bt0 = cp['tail0'][1][None, :]
    wt1 = cp['tail1'][0].T
    bt1 = cp['tail1'][1][None, :]
    wt2 = cp['tail2'][0].T
    bt2 = cp['tail2'][1][None, :]
    wg0s = gp['style0'][0]
    w0zg, w0pg = wg0s[:, :128].T, wg0s[:, 128:].T
    b0g = gp['style0'][1][None, :]
    w1g = gp['style1'][0].T
    b1g = gp['style1'][1][None, :]
    wfe = gp['fe'][0]
    wfex, wfeg = wfe[:, :CHN].T, wfe[:, CHN:].T
    bfe = gp['fe'][1][None, :]

    f32 = jnp.float32
    rows = lambda c: jax.ShapeDtypeStruct((N, c), f32)

    ws1 = [w0z, w0p, b0, w1, b1, brcat, phase,
           cb0['wh0'], cb0['bh0'], cb0['wh1'], cb0['bh1'],
           cb0['wfp'], cb0['wfx'], cb0['bf']]
    style, x0, a1, c1 = pl.pallas_call(
        _tc1_body,
        grid=(GRID,),
        in_specs=[_row_spec(3), _row_spec(CHN)] + _wspecs(ws1),
        out_specs=[_row_spec(CHN)] * 4,
        out_shape=[rows(CHN)] * 4,
    )(pos, z, *ws1)

    pt1a, pt1b = _edge_agg(a1, c1, src, dst)

    ws2 = [cb0['wg0'], cb0['bg0'], cb0['wg1'], cb0['bg1'],
           cb0['wag'], cb0['bag'], cb0['wab'], cb0['bab'],
           cb1['wh0'], cb1['bh0'], cb1['wh1'], cb1['bh1'],
           cb1['wfp'], cb1['wfx'], cb1['bf']]
    x1, a2, c2 = pl.pallas_call(
        _tc2_body,
        grid=(GRID,),
        in_specs=[_row_spec(CHN)] * 4 + [_row_spec(3)] + _wspecs(ws2),
        out_specs=[_row_spec(CHN)] * 3,
        out_shape=[rows(CHN)] * 3,
    )(x0, pt1a, pt1b, style, pos, *ws2)

    pt2a, pt2b = _edge_agg(a2, c2, src, dst)

    ws3a = [cb1['wg0'], cb1['bg0'], cb1['wg1'], cb1['bg1'],
            cb1['wag'], cb1['bag'], cb1['wab'], cb1['bab']]
    x2, gmax = pl.pallas_call(
        _tc3a_body,
        grid=(GRID,),
        in_specs=[_row_spec(CHN)] * 4 + [_row_spec(1)] + _wspecs(ws3a),
        out_specs=[_row_spec(CHN), _full_spec((2, CHN))],
        out_shape=[rows(CHN), jax.ShapeDtypeStruct((2, CHN), f32)],
    )(x1, pt2a, pt2b, style, batch2, *ws3a)

    ws3b = [wgc0, bgc0, wgc1, bgc1,
            wt0x, wt0g, bt0, wt1, bt1, wt2, bt2,
            w0zg, w0pg, b0g, w1g, b1g,
            wfex, wfeg, bfe,
            gb0['wh0'], gb0['bh0'], gb0['wh1'], gb0['bh1'],
            gb0['wfp'], gb0['wfx'], gb0['bf']]
    pc, h0, styleg, a3, c3 = pl.pallas_call(
        _tc3b_body,
        grid=(GRID,),
        in_specs=[_row_spec(CHN), _row_spec(CHN), _row_spec(1),
                  _full_spec((2, CHN))] + _wspecs(ws3b),
        out_specs=[_row_spec(3)] + [_row_spec(CHN)] * 4,
        out_shape=[rows(3)] + [rows(CHN)] * 4,
    )(x2, z, batch2, gmax, *ws3b)

    pt3a, pt3b = _edge_agg(a3, c3, src, dst)

    ws4 = [gb0['wg0'], gb0['bg0'], gb0['wg1'], gb0['bg1'],
           gb0['wag'], gb0['bag'], gb0['wab'], gb0['bab']]
    hout = pl.pallas_call(
        _tc4_body,
        grid=(GRID,),
        in_specs=[_row_spec(CHN)] * 4 + _wspecs(ws4),
        out_specs=_row_spec(CHN),
        out_shape=rows(CHN),
    )(h0, pt3a, pt3b, styleg, *ws4)

    return jnp.concatenate([pc, hout], axis=-1)
